# Initial kernel scaffold; baseline (speedup 1.0000x reference)
import jax
import jax.numpy as jnp
from jax.experimental import pallas as pl


def kernel(x, edge_index, edge_weight, W):
    raise NotImplementedError("write your pallas kernel here")


# speedup vs baseline: 4.5708x; 4.5708x over previous
import functools

import jax
import jax.numpy as jnp
from jax import lax
from jax.experimental import pallas as pl
from jax.experimental.pallas import tpu as pltpu
from jax.experimental.pallas import tpu_sc as plsc

N_NODES = 10000
N_PAD = 10240
D_FEAT = 128
N_EDGES = 320000

NC, NS, L = 2, 16, 16
NW = NC * NS
E_PER_W = N_EDGES // NW
CHUNK = 80
N_CHUNKS = E_PER_W // CHUNK
ROWS_PER_TILE = N_PAD // NS
ZROWS = 128


def _sc_aggregate(x, src, dst, w):
    mesh = plsc.VectorSubcoreMesh(core_axis_name="c", subcore_axis_name="s")

    @functools.partial(
        pl.kernel,
        out_type=jax.ShapeDtypeStruct((NC * N_PAD, D_FEAT), jnp.float32),
        mesh=mesh,
        scratch_types=[
            pltpu.VMEM((CHUNK,), jnp.int32),
            pltpu.VMEM((CHUNK,), jnp.int32),
            pltpu.VMEM((CHUNK,), jnp.float32),
            pltpu.VMEM((CHUNK, D_FEAT), jnp.float32),
            pltpu.VMEM((ZROWS, D_FEAT), jnp.float32),
            pltpu.VMEM_SHARED((N_PAD, D_FEAT), jnp.float32),
            pltpu.SemaphoreType.DMA,
        ],
    )
    def k(x_hbm, src_hbm, dst_hbm, w_hbm, out_hbm,
          src_v, dst_v, w_v, rows_v, zst_v, acc_sh, sem):
        c = lax.axis_index("c")
        s = lax.axis_index("s")
        wid = c * NS + s

        zero16 = jnp.zeros((L,), jnp.float32)

        def zrow(r, _):
            for cc in range(D_FEAT // L):
                zst_v[r, pl.ds(cc * L, L)] = zero16
            return 0

        lax.fori_loop(0, ZROWS, zrow, 0)
        for j in range(ROWS_PER_TILE // ZROWS):
            pltpu.sync_copy(
                zst_v, acc_sh.at[pl.ds(s * ROWS_PER_TILE + j * ZROWS, ZROWS), :])
        plsc.subcore_barrier()

        def chunk_body(i, _):
            base = wid * E_PER_W + i * CHUNK
            pltpu.sync_copy(src_hbm.at[pl.ds(base, CHUNK)], src_v)
            pltpu.sync_copy(dst_hbm.at[pl.ds(base, CHUNK)], dst_v)
            pltpu.sync_copy(w_hbm.at[pl.ds(base, CHUNK)], w_v)
            pltpu.async_copy(x_hbm.at[src_v], rows_v, sem).wait()

            def g_body(g, _):
                wvec = w_v[pl.ds(g * L, L)]
                for e16 in range(L):
                    wv = jnp.full((L,), wvec[e16])
                    e = g * L + e16
                    for cc in range(D_FEAT // L):
                        sl = pl.ds(cc * L, L)
                        rows_v[e, sl] = rows_v[e, sl] * wv
                return 0

            lax.fori_loop(0, CHUNK // L, g_body, 0)
            pltpu.sync_copy(rows_v, acc_sh.at[dst_v], add=True)
            return 0

        lax.fori_loop(0, N_CHUNKS, chunk_body, 0)
        plsc.subcore_barrier()

        row0 = c * N_PAD + s * ROWS_PER_TILE
        pltpu.sync_copy(acc_sh.at[pl.ds(s * ROWS_PER_TILE, ROWS_PER_TILE), :],
                        out_hbm.at[pl.ds(row0, ROWS_PER_TILE), :])

    return k(x, src, dst, w)


def _tc_finish(agg, wt):
    BLK = 1024
    assert N_PAD % BLK == 0

    def body(a0_ref, a1_ref, wt_ref, o_ref):
        sacc = a0_ref[...] + a1_ref[...]
        h = jnp.dot(sacc, wt_ref[...], preferred_element_type=jnp.float32)
        o_ref[...] = 0.5 * h * (1.0 + lax.erf(h * 0.7071067811865476))

    return pl.pallas_call(
        body,
        grid=(N_PAD // BLK,),
        in_specs=[
            pl.BlockSpec((BLK, D_FEAT), lambda i: (i, 0)),
            pl.BlockSpec((BLK, D_FEAT),
                         lambda i: (N_PAD // BLK + i, 0)),
            pl.BlockSpec((D_FEAT, D_FEAT), lambda i: (0, 0)),
        ],
        out_specs=pl.BlockSpec((BLK, D_FEAT), lambda i: (i, 0)),
        out_shape=jax.ShapeDtypeStruct((N_NODES, D_FEAT), jnp.float32),
    )(agg, agg, wt)


def kernel(x, edge_index, edge_weight, W):
    src = edge_index[1]
    dst = edge_index[0]
    agg = _sc_aggregate(x, src, dst, edge_weight)
    return _tc_finish(agg, W.T)


# speedup vs baseline: 11.2943x; 2.4710x over previous
import functools

import jax
import jax.numpy as jnp
from jax import lax
from jax.experimental import pallas as pl
from jax.experimental.pallas import tpu as pltpu
from jax.experimental.pallas import tpu_sc as plsc

N_NODES = 10000
N_PAD = 10240
D_FEAT = 128
N_EDGES = 320000

NC, NS, L = 2, 16, 16
NW = NC * NS
E_PER_W = N_EDGES // NW
CHUNK = 80
N_CHUNKS = E_PER_W // CHUNK
ROWS_PER_TILE = N_PAD // NS


def _sc_aggregate(x, src3, dst1, w1):
    mesh = plsc.VectorSubcoreMesh(core_axis_name="c", subcore_axis_name="s")

    @functools.partial(
        pl.kernel,
        out_type=jax.ShapeDtypeStruct((NC * N_PAD, D_FEAT), jnp.float32),
        mesh=mesh,
        scratch_types=[
            pltpu.VMEM((N_CHUNKS, CHUNK), jnp.int32),
            pltpu.VMEM((CHUNK, D_FEAT), jnp.float32),
            pltpu.VMEM((CHUNK, D_FEAT), jnp.float32),
            pltpu.VMEM((CHUNK,), jnp.int32),
            pltpu.VMEM((CHUNK,), jnp.int32),
            pltpu.VMEM((CHUNK,), jnp.float32),
            pltpu.VMEM((CHUNK,), jnp.float32),
            pltpu.VMEM_SHARED((N_PAD, D_FEAT), jnp.float32),
            pltpu.SemaphoreType.DMA,
            pltpu.SemaphoreType.DMA,
        ],
    )
    def k(x_hbm, src_hbm, dst_hbm, w_hbm, out_hbm,
          src_v, rows0, rows1, dst0, dst1, w0, w1, acc_sh, sem0, sem1):
        c = lax.axis_index("c")
        s = lax.axis_index("s")
        wid = c * NS + s

        pltpu.sync_copy(src_hbm.at[wid], src_v)

        zero16 = jnp.zeros((L,), jnp.float32)

        def zrow(r, _):
            for cc in range(D_FEAT // L):
                sl = pl.ds(cc * L, L)
                rows0[r, sl] = zero16
                rows1[r, sl] = zero16
            return 0

        lax.fori_loop(0, CHUNK, zrow, 0)
        for j in range(ROWS_PER_TILE // (2 * CHUNK)):
            base = s * ROWS_PER_TILE + j * 2 * CHUNK
            pltpu.sync_copy(rows0, acc_sh.at[pl.ds(base, CHUNK), :])
            pltpu.sync_copy(rows1, acc_sh.at[pl.ds(base + CHUNK, CHUNK), :])
        plsc.subcore_barrier()

        def scale_scatter(rows_v, dst_v, w_v):
            def g_body(g, _):
                wvec = w_v[pl.ds(g * L, L)]
                for e16 in range(L):
                    wv = jnp.full((L,), wvec[e16])
                    e = g * L + e16
                    for cc in range(D_FEAT // L):
                        sl = pl.ds(cc * L, L)
                        rows_v[e, sl] = rows_v[e, sl] * wv
                return 0

            lax.fori_loop(0, CHUNK // L, g_body, 0)
            pltpu.sync_copy(rows_v, acc_sh.at[dst_v], add=True)

        def fire(i, rows_v, dst_v, w_v, sem):
            base = wid * E_PER_W + i * CHUNK
            pltpu.async_copy(x_hbm.at[src_v.at[i]], rows_v, sem)
            pltpu.async_copy(dst_hbm.at[pl.ds(base, CHUNK)], dst_v, sem)
            pltpu.async_copy(w_hbm.at[pl.ds(base, CHUNK)], w_v, sem)

        def drain(i, rows_v, dst_v, w_v, sem):
            base = wid * E_PER_W + i * CHUNK
            pltpu.make_async_copy(x_hbm.at[src_v.at[i]], rows_v, sem).wait()
            pltpu.make_async_copy(
                dst_hbm.at[pl.ds(base, CHUNK)], dst_v, sem).wait()
            pltpu.make_async_copy(
                w_hbm.at[pl.ds(base, CHUNK)], w_v, sem).wait()

        fire(0, rows0, dst0, w0, sem0)

        def pair_body(j, _):
            a = 2 * j
            fire(a + 1, rows1, dst1, w1, sem1)
            drain(a, rows0, dst0, w0, sem0)
            scale_scatter(rows0, dst0, w0)
            fire(a + 2, rows0, dst0, w0, sem0)
            drain(a + 1, rows1, dst1, w1, sem1)
            scale_scatter(rows1, dst1, w1)
            return 0

        lax.fori_loop(0, (N_CHUNKS - 1) // 2, pair_body, 0)
        drain(N_CHUNKS - 1, rows0, dst0, w0, sem0)
        scale_scatter(rows0, dst0, w0)
        plsc.subcore_barrier()

        row0 = c * N_PAD + s * ROWS_PER_TILE
        pltpu.sync_copy(acc_sh.at[pl.ds(s * ROWS_PER_TILE, ROWS_PER_TILE), :],
                        out_hbm.at[pl.ds(row0, ROWS_PER_TILE), :])

    return k(x, src3, dst1, w1)


def _tc_finish(agg, wt):
    BLK = 1024
    assert N_PAD % BLK == 0

    def body(a0_ref, a1_ref, wt_ref, o_ref):
        sacc = a0_ref[...] + a1_ref[...]
        h = jnp.dot(sacc, wt_ref[...], preferred_element_type=jnp.float32)
        o_ref[...] = 0.5 * h * (1.0 + lax.erf(h * 0.7071067811865476))

    return pl.pallas_call(
        body,
        grid=(N_PAD // BLK,),
        in_specs=[
            pl.BlockSpec((BLK, D_FEAT), lambda i: (i, 0)),
            pl.BlockSpec((BLK, D_FEAT),
                         lambda i: (N_PAD // BLK + i, 0)),
            pl.BlockSpec((D_FEAT, D_FEAT), lambda i: (0, 0)),
        ],
        out_specs=pl.BlockSpec((BLK, D_FEAT), lambda i: (i, 0)),
        out_shape=jax.ShapeDtypeStruct((N_NODES, D_FEAT), jnp.float32),
    )(agg, agg, wt)


def kernel(x, edge_index, edge_weight, W):
    src3 = edge_index[1].reshape(NW, N_CHUNKS, CHUNK)
    agg = _sc_aggregate(x, src3, edge_index[0], edge_weight)
    return _tc_finish(agg, W.T)
